# baseline (speedup 1.0000x reference)
import jax
import jax.numpy as jnp
from jax.experimental import pallas as pl
from jax.experimental.pallas import tpu as pltpu

_BM = 400


def _h_kernel(x_ref, w_ref, h_ref):
    h_ref[...] = jnp.dot(x_ref[...], w_ref[...],
                         preferred_element_type=jnp.float32)


def _agg_kernel(a_ref, h_ref, b_ref, o_ref):
    acc = jnp.dot(a_ref[...], h_ref[...], preferred_element_type=jnp.float32)
    o_ref[...] = jnp.maximum(acc + b_ref[...], 0.0)


def kernel(A, x, W, b):
    N, D = x.shape
    h = pl.pallas_call(
        _h_kernel,
        out_shape=jax.ShapeDtypeStruct((N, D), jnp.float32),
    )(x, W)

    return pl.pallas_call(
        _agg_kernel,
        grid=(N // _BM,),
        in_specs=[
            pl.BlockSpec((_BM, N), lambda i: (i, 0)),
            pl.BlockSpec((N, D), lambda i: (0, 0)),
            pl.BlockSpec((1, D), lambda i: (0, 0)),
        ],
        out_specs=pl.BlockSpec((_BM, D), lambda i: (i, 0)),
        out_shape=jax.ShapeDtypeStruct((N, D), jnp.float32),
        compiler_params=pltpu.CompilerParams(
            dimension_semantics=("parallel",)),
    )(A, h, b.reshape(1, D))


# speedup vs baseline: 1.0367x; 1.0367x over previous
import jax
import jax.numpy as jnp
from jax.experimental import pallas as pl
from jax.experimental.pallas import tpu as pltpu

_BM = 400


def _fused_kernel(a_hbm, x_ref, w_ref, b_ref, o_ref, h_ref, abuf, sem):
    i = pl.program_id(0)
    nb = pl.num_programs(0)

    @pl.when(i == 0)
    def _():
        pltpu.make_async_copy(a_hbm.at[pl.ds(0, _BM), :], abuf.at[0],
                              sem.at[0]).start()

    @pl.when((i + 1 < nb) & (i % 2 == 0))
    def _():
        pltpu.make_async_copy(a_hbm.at[pl.ds((i + 1) * _BM, _BM), :],
                              abuf.at[1], sem.at[1]).start()

    @pl.when((i + 1 < nb) & (i % 2 == 1))
    def _():
        pltpu.make_async_copy(a_hbm.at[pl.ds((i + 1) * _BM, _BM), :],
                              abuf.at[0], sem.at[0]).start()

    @pl.when(i == 0)
    def _():
        h_ref[...] = jnp.dot(x_ref[...], w_ref[...],
                             preferred_element_type=jnp.float32)

    def _compute(slot):
        pltpu.make_async_copy(a_hbm.at[pl.ds(i * _BM, _BM), :],
                              abuf.at[slot], sem.at[slot]).wait()
        acc = jnp.dot(abuf[slot], h_ref[...],
                      preferred_element_type=jnp.float32)
        o_ref[...] = jnp.maximum(acc + b_ref[...], 0.0)

    @pl.when(i % 2 == 0)
    def _():
        _compute(0)

    @pl.when(i % 2 == 1)
    def _():
        _compute(1)


def kernel(A, x, W, b):
    N, D = x.shape
    return pl.pallas_call(
        _fused_kernel,
        grid=(N // _BM,),
        in_specs=[
            pl.BlockSpec(memory_space=pltpu.MemorySpace.HBM),
            pl.BlockSpec((N, D), lambda i: (0, 0)),
            pl.BlockSpec((D, D), lambda i: (0, 0)),
            pl.BlockSpec((1, D), lambda i: (0, 0)),
        ],
        out_specs=pl.BlockSpec((_BM, D), lambda i: (i, 0)),
        out_shape=jax.ShapeDtypeStruct((N, D), jnp.float32),
        scratch_shapes=[
            pltpu.VMEM((N, D), jnp.float32),
            pltpu.VMEM((2, _BM, N), jnp.float32),
            pltpu.SemaphoreType.DMA((2,)),
        ],
    )(A, x, W, b.reshape(1, D))


# speedup vs baseline: 1.0425x; 1.0057x over previous
import jax
import jax.numpy as jnp
from jax.experimental import pallas as pl
from jax.experimental.pallas import tpu as pltpu

_BM = 400


def _fused_kernel(a_ref, x_ref, w_ref, b_ref, o_ref, h_ref):
    @pl.when(pl.program_id(0) == 0)
    def _():
        h_ref[...] = jnp.dot(x_ref[...], w_ref[...],
                             preferred_element_type=jnp.float32)

    acc = jnp.dot(a_ref[...], h_ref[...], preferred_element_type=jnp.float32)
    o_ref[...] = jnp.maximum(acc + b_ref[...], 0.0)


def kernel(A, x, W, b):
    N, D = x.shape
    return pl.pallas_call(
        _fused_kernel,
        grid=(N // _BM,),
        in_specs=[
            pl.BlockSpec((_BM, N), lambda i: (i, 0)),
            pl.BlockSpec((N, D), lambda i: (0, 0)),
            pl.BlockSpec((D, D), lambda i: (0, 0)),
            pl.BlockSpec((1, D), lambda i: (0, 0)),
        ],
        out_specs=pl.BlockSpec((_BM, D), lambda i: (i, 0)),
        out_shape=jax.ShapeDtypeStruct((N, D), jnp.float32),
        scratch_shapes=[pltpu.VMEM((N, D), jnp.float32)],
    )(A, x, W, b.reshape(1, D))


# speedup vs baseline: 1.0505x; 1.0076x over previous
import jax
import jax.numpy as jnp
from jax.experimental import pallas as pl


_BM = 400


def _fused_kernel(a_ref, x_ref, w_ref, b_ref, o_ref):
    g = jnp.dot(a_ref[...], x_ref[...], preferred_element_type=jnp.float32)
    acc = jnp.dot(g, w_ref[...], preferred_element_type=jnp.float32)
    o_ref[...] = jnp.maximum(acc + b_ref[...], 0.0)


def kernel(A, x, W, b):
    N, D = x.shape
    return pl.pallas_call(
        _fused_kernel,
        grid=(N // _BM,),
        in_specs=[
            pl.BlockSpec((_BM, N), lambda i: (i, 0)),
            pl.BlockSpec((N, D), lambda i: (0, 0)),
            pl.BlockSpec((D, D), lambda i: (0, 0)),
            pl.BlockSpec((1, D), lambda i: (0, 0)),
        ],
        out_specs=pl.BlockSpec((_BM, D), lambda i: (i, 0)),
        out_shape=jax.ShapeDtypeStruct((N, D), jnp.float32),
    )(A, x, W, b.reshape(1, D))
